# baseline (speedup 1.0000x reference)
import jax
import jax.numpy as jnp
from jax.experimental import pallas as pl
from jax.experimental.pallas import tpu as pltpu

B, N, D = 16, 2048, 2048
R = 512
K = N // R


def _merge_two_smallest(m1, m2, c1, c2):
    return jnp.minimum(m1, c1), jnp.minimum(jnp.maximum(m1, c1),
                                            jnp.minimum(m2, c2))


def _body(x_ref, w_ref, w1t_ref, b1_ref, w2t_ref, b2_ref, w3t_ref, b3_ref,
          o_ref, carry):
    i = pl.program_id(0)
    k = pl.program_id(1)

    @pl.when(k == 0)
    def _init():
        carry[0] = jnp.inf
        carry[1] = jnp.inf
        carry[2] = -jnp.inf
        carry[3] = -jnp.inf

    s = jax.lax.dot_general(
        x_ref[...], w_ref[...],
        dimension_numbers=(((1,), (0,)), ((), ())),
        preferred_element_type=jnp.float32,
    )
    iota = jax.lax.broadcasted_iota(jnp.int32, (R, 1), 0)

    cmax1 = jnp.max(s)
    idx_max = jnp.min(jnp.where(s == cmax1, iota, R))
    cmax2 = jnp.max(jnp.where(iota == idx_max, -jnp.inf, s))

    cmin1 = jnp.min(s)
    idx_min = jnp.min(jnp.where(s == cmin1, iota, R))
    cmin2 = jnp.min(jnp.where(iota == idx_min, jnp.inf, s))

    min1, min2 = _merge_two_smallest(carry[0], carry[1], cmin1, cmin2)
    nmax1, nmax2 = _merge_two_smallest(-carry[2], -carry[3], -cmax1, -cmax2)
    max1, max2 = -nmax1, -nmax2
    carry[0] = min1
    carry[1] = min2
    carry[2] = max1
    carry[3] = max2

    @pl.when(k == K - 1)
    def _head():
        h = (b1_ref[...]
             + min1 * w1t_ref[0:1, :]
             + min2 * w1t_ref[1:2, :]
             + max1 * w1t_ref[2:3, :]
             + max2 * w1t_ref[3:4, :])
        h = jax.nn.sigmoid(h)

        h2 = jax.nn.sigmoid(
            jax.lax.dot_general(h, w2t_ref[...],
                                dimension_numbers=(((1,), (0,)), ((), ())),
                                preferred_element_type=jnp.float32)
            + b2_ref[...])

        o_ref[pl.ds(i, 1), :] = jax.nn.sigmoid(
            jax.lax.dot_general(h2, w3t_ref[...],
                                dimension_numbers=(((1,), (0,)), ((), ())),
                                preferred_element_type=jnp.float32)
            + b3_ref[...])


@jax.jit
def kernel(x, W_embed, W1, b1, W2, b2, W3, b3):
    xf = x.reshape(B * N, D)
    wt = W_embed.reshape(D, 1)
    const = lambda i, k: (0, 0)

    out = pl.pallas_call(
        _body,
        grid=(B, K),
        in_specs=[
            pl.BlockSpec((R, D), lambda i, k: (i * K + k, 0)),
            pl.BlockSpec((D, 1), const),
            pl.BlockSpec((4, 200), const),
            pl.BlockSpec((1, 200), const),
            pl.BlockSpec((200, 100), const),
            pl.BlockSpec((1, 100), const),
            pl.BlockSpec((100, 1), const),
            pl.BlockSpec((1, 1), const),
        ],
        out_specs=pl.BlockSpec((B, 1), const),
        out_shape=jax.ShapeDtypeStruct((B, 1), jnp.float32),
        scratch_shapes=[pltpu.SMEM((4,), jnp.float32)],
    )(xf, wt, W1.T, b1.reshape(1, 200), W2.T, b2.reshape(1, 100),
      W3.T, b3.reshape(1, 1))
    return out.reshape(-1)
